# baseline (speedup 1.0000x reference)
import functools

import jax
import jax.numpy as jnp
from jax import lax
from jax.experimental import pallas as pl
from jax.experimental.pallas import tpu as pltpu
from jax.experimental.pallas import tpu_sc as plsc

_DIM = 64
_IDX_CHUNK = 128


def kernel(user_indices, item_indices, user_table, item_table, W, b):
    B = user_indices.shape[0]
    info = plsc.get_sparse_core_info()
    NC, NS = info.num_cores, info.num_subcores
    NW = NC * NS
    b_per_w = B // NW
    n_chunks = b_per_w // _IDX_CHUNK

    ui = user_indices.astype(jnp.int32).reshape(NW, n_chunks, _IDX_CHUNK)
    ii = item_indices.astype(jnp.int32).reshape(NW, n_chunks, _IDX_CHUNK)
    wb = jnp.concatenate([W[:, 0], jnp.full((16,), b[0], jnp.float32)])

    mesh = plsc.VectorSubcoreMesh(core_axis_name="c", subcore_axis_name="s")

    @functools.partial(
        pl.kernel,
        mesh=mesh,
        out_type=jax.ShapeDtypeStruct((B,), jnp.float32),
        compiler_params=pltpu.CompilerParams(
            needs_layout_passes=False, use_tc_tiling_on_sc=False),
        scratch_types=[
            pltpu.VMEM((n_chunks, _IDX_CHUNK), jnp.int32),
            pltpu.VMEM((n_chunks, _IDX_CHUNK), jnp.int32),
            pltpu.VMEM((b_per_w, _DIM), jnp.float32),
            pltpu.VMEM((b_per_w, _DIM), jnp.float32),
            pltpu.VMEM((80,), jnp.float32),
            pltpu.VMEM((b_per_w,), jnp.float32),
            pltpu.VMEM((256,), jnp.float32),
            pltpu.SemaphoreType.DMA,
        ],
    )
    def gmf(ui_hbm, ii_hbm, ut_hbm, it_hbm, wb_hbm, out_hbm,
            idx_u, idx_i, rows_u, rows_v, wv, out_v, tpose, sem):
        wid = lax.axis_index("s") * NC + lax.axis_index("c")
        base = wid * b_per_w

        pltpu.sync_copy(ui_hbm.at[wid], idx_u)
        pltpu.sync_copy(ii_hbm.at[wid], idx_i)
        pltpu.sync_copy(wb_hbm, wv)

        handles = []
        for j in range(n_chunks):
            handles.append(pltpu.async_copy(
                ut_hbm.at[idx_u.at[j]],
                rows_u.at[pl.ds(j * _IDX_CHUNK, _IDX_CHUNK)], sem))
            handles.append(pltpu.async_copy(
                it_hbm.at[idx_i.at[j]],
                rows_v.at[pl.ds(j * _IDX_CHUNK, _IDX_CHUNK)], sem))
        for h in handles:
            h.wait()

        wc = [wv[pl.ds(c * 16, 16)] for c in range(_DIM // 16)]
        bias = wv[pl.ds(_DIM, 16)]
        lane = lax.iota(jnp.int32, 16)
        col0 = lane * 16

        def body(t, carry):
            r0 = t * 16
            for j in range(16):
                s = None
                for c in range(_DIM // 16):
                    u = rows_u[r0 + j, pl.ds(c * 16, 16)]
                    v = rows_v[r0 + j, pl.ds(c * 16, 16)]
                    term = u * v * wc[c]
                    s = term if s is None else s + term
                tpose[pl.ds(j * 16, 16)] = s
            acc = bias
            for j in range(16):
                acc = acc + plsc.load_gather(tpose, [col0 + j])
            out_v[pl.ds(r0, 16)] = acc
            return carry

        lax.fori_loop(0, b_per_w // 16, body, 0)
        pltpu.sync_copy(out_v, out_hbm.at[pl.ds(base, b_per_w)])

    out = gmf(ui, ii, user_table, item_table, wb)
    return out.reshape(B, 1)


# speedup vs baseline: 1.5418x; 1.5418x over previous
import functools

import jax
import jax.numpy as jnp
from jax import lax
from jax.experimental import pallas as pl
from jax.experimental.pallas import tpu as pltpu
from jax.experimental.pallas import tpu_sc as plsc

_DIM = 64
_G = 16
_NBUF = 4


def kernel(user_indices, item_indices, user_table, item_table, W, b):
    B = user_indices.shape[0]
    info = plsc.get_sparse_core_info()
    NC, NS = info.num_cores, info.num_subcores
    NW = NC * NS
    b_per_w = B // NW
    n_groups = b_per_w // _G

    ui = user_indices.astype(jnp.int32).reshape(NW, n_groups, _G)
    ii = item_indices.astype(jnp.int32).reshape(NW, n_groups, _G)
    wb = jnp.concatenate([W[:, 0], jnp.full((_G,), b[0], jnp.float32)])

    mesh = plsc.VectorSubcoreMesh(core_axis_name="c", subcore_axis_name="s")

    @functools.partial(
        pl.kernel,
        mesh=mesh,
        out_type=jax.ShapeDtypeStruct((B,), jnp.float32),
        compiler_params=pltpu.CompilerParams(needs_layout_passes=False),
        scratch_types=[
            pltpu.VMEM((n_groups, _G), jnp.int32),
            pltpu.VMEM((n_groups, _G), jnp.int32),
            pltpu.VMEM((_NBUF * _G, _DIM), jnp.float32),
            pltpu.VMEM((_NBUF * _G, _DIM), jnp.float32),
            pltpu.VMEM((_DIM + _G,), jnp.float32),
            pltpu.VMEM((b_per_w,), jnp.float32),
            pltpu.VMEM((_G * _G,), jnp.float32),
            pltpu.SemaphoreType.DMA,
            pltpu.SemaphoreType.DMA,
            pltpu.SemaphoreType.DMA,
            pltpu.SemaphoreType.DMA,
            pltpu.SemaphoreType.DMA,
            pltpu.SemaphoreType.DMA,
            pltpu.SemaphoreType.DMA,
            pltpu.SemaphoreType.DMA,
        ],
    )
    def gmf(ui_hbm, ii_hbm, ut_hbm, it_hbm, wb_hbm, out_hbm,
            idx_u, idx_i, urows, vrows, wv, out_v, tpose, *sems):
        usems, vsems = sems[:_NBUF], sems[_NBUF:]
        wid = lax.axis_index("s") * NC + lax.axis_index("c")

        pltpu.sync_copy(ui_hbm.at[wid], idx_u)
        pltpu.sync_copy(ii_hbm.at[wid], idx_i)
        pltpu.sync_copy(wb_hbm, wv)

        wc = [wv[pl.ds(c * 16, 16)] for c in range(_DIM // 16)]
        bias = wv[pl.ds(_DIM, _G)]
        lane = lax.iota(jnp.int32, 16)
        col0 = lane * 16

        def issue(g, slot):
            uvec = idx_u[g, pl.ds(0, _G)]
            ivec = idx_i[g, pl.ds(0, _G)]
            for j in range(_G):
                pltpu.async_copy(ut_hbm.at[uvec[j]],
                                 urows.at[slot * _G + j], usems[slot])
                pltpu.async_copy(it_hbm.at[ivec[j]],
                                 vrows.at[slot * _G + j], vsems[slot])

        def drain(slot):
            for j in range(_G):
                pltpu.make_async_copy(
                    ut_hbm.at[0], urows.at[slot * _G + j], usems[slot]).wait()
                pltpu.make_async_copy(
                    it_hbm.at[0], vrows.at[slot * _G + j], vsems[slot]).wait()

        def compute(g, slot):
            for j in range(_G):
                s = None
                for c in range(_DIM // 16):
                    u = urows[slot * _G + j, pl.ds(c * 16, 16)]
                    v = vrows[slot * _G + j, pl.ds(c * 16, 16)]
                    term = u * v * wc[c]
                    s = term if s is None else s + term
                tpose[pl.ds(j * 16, 16)] = s
            acc = bias
            for j in range(_G):
                acc = acc + plsc.load_gather(tpose, [col0 + j])
            out_v[pl.ds(g * _G, _G)] = acc

        for slot in range(_NBUF):
            issue(slot, slot)

        def body(k, carry):
            for slot in range(_NBUF):
                g = k * _NBUF + slot
                drain(slot)
                compute(g, slot)

                @pl.when(g + _NBUF < n_groups)
                def _():
                    issue(g + _NBUF, slot)
            return carry

        lax.fori_loop(0, n_groups // _NBUF, body, 0)
        pltpu.sync_copy(out_v, out_hbm.at[pl.ds(wid * b_per_w, b_per_w)])

    out = gmf(ui, ii, user_table, item_table, wb)
    return out.reshape(B, 1)
